# Initial kernel scaffold; baseline (speedup 1.0000x reference)
import jax
import jax.numpy as jnp
from jax.experimental import pallas as pl


def kernel(x, edge_index, counts, use_counts, batch, emb, cW1, cb1, cW2, cb2, conv_W1, conv_b1, conv_W2, conv_b2, conv_gamma, conv_beta, dW1, db1, dW2, db2):
    raise NotImplementedError("write your pallas kernel here")


# speedup vs baseline: 4.4880x; 4.4880x over previous
import functools

import jax
import jax.numpy as jnp
from jax import lax
from jax.experimental import pallas as pl
from jax.experimental.pallas import tpu as pltpu
from jax.experimental.pallas import tpu_sc as plsc

N = 10000
E = 320000
HID = 128
CNT = 16
NUM_LAYERS = 4
NUM_EMB = 121
MAX_NODES = 121
NUM_GRAPHS = 100

F32 = jnp.float32


_NC = 2
_NS = 16
_K = 80
_EPC = E // _NC
_EPT = _EPC // _NS
_NCH = _EPT // _K
_ZR = 80
_NZCH = N // _ZR
_ZITER = (_NZCH + _NS - 1) // _NS


def _make_segsum(D):
  mesh = plsc.VectorSubcoreMesh(core_axis_name="c", subcore_axis_name="s")

  @functools.partial(
      pl.kernel,
      mesh=mesh,
      out_type=[
          jax.ShapeDtypeStruct((N, D), F32),
          jax.ShapeDtypeStruct((N, D), F32),
      ],
      scratch_types=[
          pltpu.VMEM((_K,), jnp.int32),
          pltpu.VMEM((_K,), jnp.int32),
          pltpu.VMEM((_K, D), F32),
          pltpu.VMEM((_ZR, D), F32),
          pltpu.VMEM_SHARED((N, D), F32),
          pltpu.SemaphoreType.DMA,
      ],
  )
  def segsum(p_hbm, src_hbm, dst_hbm, agg0_hbm, agg1_hbm,
             srcv, dstv, rows, bounce, acc, sem):
    c = lax.axis_index("c")
    s = lax.axis_index("s")

    def _zrow(r, _):
      def _zcol(j, _):
        bounce[r, pl.ds(j * 16, 16)] = jnp.zeros((16,), F32)
        return 0
      lax.fori_loop(0, D // 16, _zcol, 0)
      return 0
    lax.fori_loop(0, _ZR, _zrow, 0)

    def _zchunk(j, _):
      idx = s + j * _NS
      @pl.when(idx < _NZCH)
      def _():
        pltpu.sync_copy(bounce, acc.at[pl.ds(idx * _ZR, _ZR)])
      return 0
    lax.fori_loop(0, _ZITER, _zchunk, 0)
    plsc.subcore_barrier()

    base_e = (c * _NS + s) * _EPT

    def _step(k, _):
      off = base_e + k * _K
      pltpu.sync_copy(src_hbm.at[pl.ds(off, _K)], srcv)
      pltpu.sync_copy(dst_hbm.at[pl.ds(off, _K)], dstv)
      pltpu.async_copy(p_hbm.at[srcv], rows, sem).wait()
      pltpu.sync_copy(rows, acc.at[dstv], add=True)
      return 0
    lax.fori_loop(0, _NCH, _step, 0)
    plsc.subcore_barrier()

    def _writeout(out_hbm):
      def _w(j, _):
        idx = s + j * _NS
        @pl.when(idx < _NZCH)
        def _():
          r0 = idx * _ZR
          pltpu.sync_copy(acc.at[pl.ds(r0, _ZR)], bounce)
          pltpu.sync_copy(bounce, out_hbm.at[pl.ds(r0, _ZR)])
        return 0
      lax.fori_loop(0, _ZITER, _w, 0)

    @pl.when(c == 0)
    def _():
      _writeout(agg0_hbm)

    @pl.when(c == 1)
    def _():
      _writeout(agg1_hbm)

  return segsum


_segsum_cache = {}


def _segsum(p, src, dst):
  D = p.shape[1]
  if D not in _segsum_cache:
    _segsum_cache[D] = _make_segsum(D)
  return _segsum_cache[D](p, src, dst)


def _enc_body(x_ref, counts_ref, uc_ref, emb_ref, cW1_ref, cb1_ref,
              cW2_ref, cb2_ref, W1a_ref, W1b_ref, out_ref):
  xi = x_ref[...]
  onehot = (lax.broadcasted_iota(jnp.int32, (N, NUM_EMB), 1) == xi
            ).astype(F32)
  he = jnp.dot(onehot, emb_ref[...], preferred_element_type=F32)
  ch = jnp.maximum(
      jnp.dot(counts_ref[...], cW1_ref[...], preferred_element_type=F32)
      + cb1_ref[...], 0.0)
  ch = jnp.dot(ch, cW2_ref[...], preferred_element_type=F32) + cb2_ref[...]
  ch = ch * uc_ref[0, 0]
  out_ref[...] = (jnp.dot(he, W1a_ref[...], preferred_element_type=F32)
                  + jnp.dot(ch, W1b_ref[...], preferred_element_type=F32))


def _layer_body(*refs, residual, last):
  if residual:
    h_ref, p_ref, a0_ref, a1_ref, b1_ref, W2_ref, b2_ref, g_ref, be_ref = \
        refs[:9]
    rest = refs[9:]
  else:
    p_ref, a0_ref, a1_ref, b1_ref, W2_ref, b2_ref, g_ref, be_ref = refs[:8]
    rest = refs[8:]
  if last:
    (out_ref,) = rest
  else:
    W1n_ref, out_ref, pout_ref = rest

  z = jnp.maximum(p_ref[...] + a0_ref[...] + a1_ref[...] + b1_ref[...], 0.0)
  z = jnp.dot(z, W2_ref[...], preferred_element_type=F32) + b2_ref[...]
  mu = jnp.mean(z, axis=0, keepdims=True)
  var = jnp.mean(jnp.square(z - mu), axis=0, keepdims=True)
  z = g_ref[...] * (z - mu) * lax.rsqrt(var + 1e-5) + be_ref[...]
  z = jnp.maximum(z, 0.0)
  if residual:
    z = z + h_ref[...]
  out_ref[...] = z
  if not last:
    pout_ref[...] = jnp.dot(z, W1n_ref[...], preferred_element_type=F32)


def _final_body(h_ref, batch_ref, dW1_ref, db1_ref, dW2_ref, db2_ref,
                out_ref):
  h = h_ref[...]
  z = jnp.maximum(
      jnp.dot(h, dW1_ref[...], preferred_element_type=F32) + db1_ref[...],
      0.0)
  z = jnp.dot(z, dW2_ref[...], preferred_element_type=F32) + db2_ref[...]
  preds = 1.0 / (1.0 + jnp.exp(-z))

  b = batch_ref[...]
  Bh = (lax.broadcasted_iota(jnp.int32, (N, NUM_GRAPHS), 1) == b
        ).astype(F32)
  cnts = jnp.sum(Bh, axis=0, keepdims=True)
  tri = (lax.broadcasted_iota(jnp.int32, (NUM_GRAPHS, NUM_GRAPHS), 0)
         < lax.broadcasted_iota(jnp.int32, (NUM_GRAPHS, NUM_GRAPHS), 1)
         ).astype(F32)
  offs = jnp.dot(cnts, tri, preferred_element_type=F32)
  off_node = lax.dot_general(Bh, offs, (((1,), (1,)), ((), ())),
                             preferred_element_type=F32)
  rowid = lax.broadcasted_iota(jnp.int32, (N, 1), 0).astype(F32)
  pos = rowid - off_node
  mask = pos < float(MAX_NODES)
  Pm = ((lax.broadcasted_iota(jnp.int32, (N, MAX_NODES), 1).astype(F32)
         == pos) & mask).astype(F32)
  out = lax.dot_general(Bh, Pm * preds, (((0,), (0,)), ((), ())),
                        preferred_element_type=F32)
  out_ref[...] = out


def _tc_call(body, out_shape):
  return pl.pallas_call(body, out_shape=out_shape)


def kernel(x, edge_index, counts, use_counts, batch, emb, cW1, cb1, cW2, cb2,
           conv_W1, conv_b1, conv_W2, conv_b2, conv_gamma, conv_beta,
           dW1, db1, dW2, db2):
  x2 = x.reshape(N, 1)
  batch2 = batch.reshape(N, 1)
  uc = jnp.asarray(use_counts, F32).reshape(1, 1)
  src = edge_index[0]
  dst = edge_index[1]
  W1a = conv_W1[0][:HID]
  W1b = conv_W1[0][HID:]

  p = _tc_call(_enc_body, jax.ShapeDtypeStruct((N, HID), F32))(
      x2, counts, uc, emb, cW1, cb1.reshape(1, -1), cW2, cb2.reshape(1, -1),
      W1a, W1b)

  h = None
  for i in range(NUM_LAYERS):
    agg0, agg1 = _segsum(p, src, dst)
    residual = i > 0
    last = i == NUM_LAYERS - 1
    body = functools.partial(_layer_body, residual=residual, last=last)
    if last:
      out_shape = jax.ShapeDtypeStruct((N, HID), F32)
    else:
      out_shape = (jax.ShapeDtypeStruct((N, HID), F32),
                   jax.ShapeDtypeStruct((N, HID), F32))
    args = []
    if residual:
      args.append(h)
    args += [p, agg0, agg1, conv_b1[i].reshape(1, -1), conv_W2[i],
             conv_b2[i].reshape(1, -1), conv_gamma[i].reshape(1, -1),
             conv_beta[i].reshape(1, -1)]
    if not last:
      args.append(conv_W1[i + 1])
      h, p = _tc_call(body, out_shape)(*args)
    else:
      h = _tc_call(body, out_shape)(*args)

  out = _tc_call(_final_body,
                 jax.ShapeDtypeStruct((NUM_GRAPHS, MAX_NODES), F32))(
      h, batch2, dW1, db1.reshape(1, -1), dW2, db2.reshape(1, -1))
  return out
